# baseline (speedup 1.0000x reference)
import jax
import jax.numpy as jnp
from jax import lax
from jax.experimental import pallas as pl
from jax.experimental.pallas import tpu as pltpu
from jax.experimental.pallas import tpu_sc as plsc

NC = 2
NS = 16
L = 16
NW = NC * NS

B = 16384
D = 32
PACK = 128 // D
VR = 1000000 // PACK
BPW = B // NW
CHUNK = 128
NCHUNK = BPW // CHUNK
NGROUP = BPW // L


def _sc_kernel(idx6, event_p, edgetype_p, attrib_p, out_hbm,
               idx_v, row_v, ph, pr, pt, nh, nr, nt, out_v, sem):
    wid = lax.axis_index("s") * NC + lax.axis_index("c")
    base = wid * BPW

    pltpu.sync_copy(idx6.at[:, wid], idx_v)

    def shift_rows(j):
        def body(k, _):
            s = pl.ds(k * L, L)
            row_v[j, s] = lax.shift_right_logical(idx_v[j, s], 2)
            return 0
        lax.fori_loop(0, BPW // L, body, 0)

    for j in range(6):
        shift_rows(j)

    tables = (event_p, edgetype_p, attrib_p,
              event_p, edgetype_p, attrib_p)
    bufs = (ph, pr, pt, nh, nr, nt)

    lane = lax.iota(jnp.int32, L)
    zeros = jnp.zeros((L,), jnp.float32)
    ones = jnp.full((L,), 1.0, jnp.float32)

    for c in range(NCHUNK):
        copies = []
        for j in range(6):
            cp = pltpu.make_async_copy(
                tables[j].at[row_v.at[j, pl.ds(c * CHUNK, CHUNK)]],
                bufs[j],
                sem,
            )
            cp.start()
            copies.append(cp)
        for cp in copies:
            cp.wait()

        def margin_diff(b, offs):
            hp, rp, tp, hn, rn, tn = bufs

            def half(h):
                so = [pl.ds(offs[j] + h, L) for j in range(6)]
                d0 = jnp.abs(hp[b, so[0]] + rp[b, so[1]] - tp[b, so[2]])
                d1 = jnp.abs(hn[b, so[3]] + rn[b, so[4]] - tn[b, so[5]])
                return d0 - d1

            return jnp.sum(half(0) + half(L))

        def group_body(g, _):
            offv = [
                (idx_v[j, pl.ds(c * CHUNK + g * L, L)] & 3) * D
                for j in range(6)
            ]
            vloss = zeros
            for u in range(L):
                sc = margin_diff(g * L + u, [offv[j][u] for j in range(6)])
                vloss = jnp.where(lane == u, lax.broadcast(sc, (L,)), vloss)
            out_v[pl.ds(c * CHUNK + g * L, L)] = jnp.maximum(
                zeros, vloss + ones
            )
            return 0

        lax.fori_loop(0, CHUNK // L, group_body, 0)

    pltpu.sync_copy(out_v, out_hbm.at[pl.ds(base, BPW)])


@jax.jit
def _run(idx6, event_p, edgetype_p, attrib_p):
    mesh = plsc.VectorSubcoreMesh(core_axis_name="c", subcore_axis_name="s")
    return pl.kernel(
        _sc_kernel,
        out_type=jax.ShapeDtypeStruct((B,), jnp.float32),
        mesh=mesh,
        compiler_params=pltpu.CompilerParams(
            needs_layout_passes=False, use_tc_tiling_on_sc=False
        ),
        scratch_types=[
            pltpu.VMEM((6, BPW), jnp.int32),
            pltpu.VMEM((6, BPW), jnp.int32),
            pltpu.VMEM((CHUNK, 128), jnp.float32),
            pltpu.VMEM((CHUNK, 128), jnp.float32),
            pltpu.VMEM((CHUNK, 128), jnp.float32),
            pltpu.VMEM((CHUNK, 128), jnp.float32),
            pltpu.VMEM((CHUNK, 128), jnp.float32),
            pltpu.VMEM((CHUNK, 128), jnp.float32),
            pltpu.VMEM((BPW,), jnp.float32),
            pltpu.SemaphoreType.DMA,
        ],
    )(idx6, event_p, edgetype_p, attrib_p)


def kernel(pos_triplets, neg_triplets, event_em, edgetype_em, attrib_em):
    idx6 = jnp.concatenate(
        [pos_triplets.T, neg_triplets.T], axis=0
    ).reshape(6, NW, BPW)
    return _run(
        idx6,
        event_em.reshape(VR, 128),
        edgetype_em.reshape(VR, 128),
        attrib_em.reshape(VR, 128),
    )


# speedup vs baseline: 1.0114x; 1.0114x over previous
import jax
import jax.numpy as jnp
from jax import lax
from jax.experimental import pallas as pl
from jax.experimental.pallas import tpu as pltpu
from jax.experimental.pallas import tpu_sc as plsc

NC = 2
NS = 16
L = 16
NW = NC * NS

B = 16384
D = 32
BPW = B // NW
CHUNK = 128
NCHUNK = BPW // CHUNK
NGROUP = BPW // L


def _sc_kernel(idx6, event_em, edgetype_em, attrib_em, out_hbm,
               idx_v, ph, pr, pt, nh, nr, nt, out_v, sem):
    wid = lax.axis_index("s") * NC + lax.axis_index("c")
    base = wid * BPW

    pltpu.sync_copy(idx6.at[:, wid], idx_v)

    tables = (event_em, edgetype_em, attrib_em,
              event_em, edgetype_em, attrib_em)
    bufs = (ph, pr, pt, nh, nr, nt)

    copies = []
    for j in range(6):
        for c in range(NCHUNK):
            cp = pltpu.make_async_copy(
                tables[j].at[idx_v.at[j, pl.ds(c * CHUNK, CHUNK)]],
                bufs[j].at[pl.ds(c * CHUNK, CHUNK), :],
                sem,
            )
            cp.start()
            copies.append(cp)
    for cp in copies:
        cp.wait()

    def margin_diff(b):
        s0 = pl.ds(0, L)
        s1 = pl.ds(L, L)
        dp = jnp.abs(ph[b, s0] + pr[b, s0] - pt[b, s0]) + jnp.abs(
            ph[b, s1] + pr[b, s1] - pt[b, s1]
        )
        dn = jnp.abs(nh[b, s0] + nr[b, s0] - nt[b, s0]) + jnp.abs(
            nh[b, s1] + nr[b, s1] - nt[b, s1]
        )
        return jnp.sum(dp - dn)

    lane = lax.iota(jnp.int32, L)
    zeros = jnp.zeros((L,), jnp.float32)
    ones = jnp.full((L,), 1.0, jnp.float32)

    def group_body(g, _):
        vloss = zeros
        for u in range(L):
            sc = margin_diff(g * L + u)
            vloss = jnp.where(lane == u, lax.broadcast(sc, (L,)), vloss)
        out_v[pl.ds(g * L, L)] = jnp.maximum(zeros, vloss + ones)
        return 0

    lax.fori_loop(0, NGROUP, group_body, 0)

    pltpu.sync_copy(out_v, out_hbm.at[pl.ds(base, BPW)])


@jax.jit
def _run(idx6, event_em, edgetype_em, attrib_em):
    mesh = plsc.VectorSubcoreMesh(core_axis_name="c", subcore_axis_name="s")
    return pl.kernel(
        _sc_kernel,
        out_type=jax.ShapeDtypeStruct((B,), jnp.float32),
        mesh=mesh,
        compiler_params=pltpu.CompilerParams(
            needs_layout_passes=False, use_tc_tiling_on_sc=False
        ),
        scratch_types=[
            pltpu.VMEM((6, BPW), jnp.int32),
            pltpu.VMEM((BPW, D), jnp.float32),
            pltpu.VMEM((BPW, D), jnp.float32),
            pltpu.VMEM((BPW, D), jnp.float32),
            pltpu.VMEM((BPW, D), jnp.float32),
            pltpu.VMEM((BPW, D), jnp.float32),
            pltpu.VMEM((BPW, D), jnp.float32),
            pltpu.VMEM((BPW,), jnp.float32),
            pltpu.SemaphoreType.DMA,
        ],
    )(idx6, event_em, edgetype_em, attrib_em)


def kernel(pos_triplets, neg_triplets, event_em, edgetype_em, attrib_em):
    idx6 = jnp.concatenate(
        [pos_triplets.T, neg_triplets.T], axis=0
    ).reshape(6, NW, BPW)
    return _run(idx6, event_em, edgetype_em, attrib_em)
